# baseline (speedup 1.0000x reference)
import jax
import jax.numpy as jnp
from jax import lax
from jax.experimental import pallas as pl
from jax.experimental.pallas import tpu as pltpu
from jax.experimental.pallas import tpu_sc as plsc

D_MODEL = 1024
NUM = 16384

NUM_CORES = 2
NUM_SUBCORES = 16
NUM_WORKERS = NUM_CORES * NUM_SUBCORES

B_PER_W = NUM // NUM_WORKERS
CHUNK = 32
NCHUNKS = B_PER_W // CHUNK


def _gather_body(table_hbm, idx_hbm, out_hbm, idx_v, rows_v, sem0, sem1):
    wid = lax.axis_index("s") * NUM_CORES + lax.axis_index("c")
    base = wid * B_PER_W
    sems = (sem0, sem1)

    pltpu.sync_copy(idx_hbm.at[pl.ds(base, B_PER_W)], idx_v)

    def start(g, b):
        pltpu.async_copy(
            table_hbm.at[idx_v.at[pl.ds(g * CHUNK, CHUNK)]], rows_v.at[b],
            sems[b],
        )

    def finish(g, b):
        pltpu.make_async_copy(
            table_hbm.at[pl.ds(0, CHUNK)], rows_v.at[b], sems[b]
        ).wait()
        pltpu.sync_copy(rows_v.at[b], out_hbm.at[pl.ds(base + g * CHUNK, CHUNK)])

    start(0, 0)

    def body(i, carry):
        g0 = i * 2
        for b in range(2):
            g = g0 + b

            @pl.when(g + 1 < NCHUNKS)
            def _():
                start(g + 1, 1 - b)

            finish(g, b)
        return carry

    lax.fori_loop(0, NCHUNKS // 2, body, 0)


@jax.jit
def _positional_gather(encoding, idx):
    kernel_fn = pl.kernel(
        _gather_body,
        out_type=jax.ShapeDtypeStruct((NUM, D_MODEL), jnp.float32),
        mesh=plsc.VectorSubcoreMesh(core_axis_name="c", subcore_axis_name="s"),
        compiler_params=pltpu.CompilerParams(use_tc_tiling_on_sc=False),
        scratch_types=[
            pltpu.VMEM((B_PER_W,), jnp.int32),
            pltpu.VMEM((2, CHUNK, D_MODEL), jnp.float32),
            pltpu.SemaphoreType.DMA,
            pltpu.SemaphoreType.DMA,
        ],
    )
    return kernel_fn(encoding, idx)


def kernel(encoding, t):
    idx = t.reshape(NUM).astype(jnp.int32)
    return _positional_gather(encoding, idx)


# speedup vs baseline: 2.4217x; 2.4217x over previous
import jax
import jax.numpy as jnp
from jax import lax
from jax.experimental import pallas as pl
from jax.experimental.pallas import tpu as pltpu
from jax.experimental.pallas import tpu_sc as plsc

D_MODEL = 1024
NUM = 16384

NUM_CORES = 2
NUM_SUBCORES = 16
NUM_WORKERS = NUM_CORES * NUM_SUBCORES

B_PER_W = NUM // NUM_WORKERS
CHUNK = 32
NCHUNKS = B_PER_W // CHUNK


def _gather_body(table_hbm, idx_hbm, out_hbm, idx_v, rows_v, sem0, sem1):
    wid = lax.axis_index("s") * NUM_CORES + lax.axis_index("c")
    base = wid * B_PER_W
    sems = (sem0, sem1)

    pltpu.sync_copy(idx_hbm.at[pl.ds(base, B_PER_W)], idx_v)

    def start(g, b):
        pltpu.async_copy(
            table_hbm.at[idx_v.at[pl.ds(g * CHUNK, CHUNK)]], rows_v.at[b],
            sems[b],
        )

    def finish(g, b):
        pltpu.make_async_copy(
            table_hbm.at[pl.ds(0, CHUNK)], rows_v.at[b], sems[b]
        ).wait()
        pltpu.sync_copy(rows_v.at[b], out_hbm.at[pl.ds(base + g * CHUNK, CHUNK)])

    start(0, 0)

    def body(i, carry):
        g0 = i * 2
        for b in range(2):
            g = g0 + b

            @pl.when(g + 1 < NCHUNKS)
            def _():
                start(g + 1, 1 - b)

            finish(g, b)
        return carry

    lax.fori_loop(0, NCHUNKS // 2, body, 0)


@jax.jit
def _positional_gather(encoding, idx):
    kernel_fn = pl.kernel(
        _gather_body,
        out_type=jax.ShapeDtypeStruct((NUM, D_MODEL), jnp.float32),
        mesh=plsc.VectorSubcoreMesh(core_axis_name="c", subcore_axis_name="s"),
        scratch_types=[
            pltpu.VMEM((B_PER_W,), jnp.int32),
            pltpu.VMEM((2, CHUNK, D_MODEL), jnp.float32),
            pltpu.SemaphoreType.DMA,
            pltpu.SemaphoreType.DMA,
        ],
    )
    return kernel_fn(encoding, idx)


def kernel(encoding, t):
    idx = t.reshape(NUM).astype(jnp.int32)
    return _positional_gather(encoding, idx)
